# baseline (speedup 1.0000x reference)
import functools

import jax
import jax.numpy as jnp
from jax import lax
from jax.experimental import pallas as pl
from jax.experimental.pallas import tpu as pltpu
from jax.experimental.pallas import tpu_sc as plsc

_L = 16


@functools.lru_cache(maxsize=None)
def _build(B, D, NUM, n_rows_a):
    info = plsc.get_sparse_core_info()
    NC, NS = info.num_cores, info.num_subcores
    NW = NC * NS
    assert B % (8 * NW) == 0 and D == _L
    n_per_w = B // NW

    mesh = plsc.VectorSubcoreMesh(core_axis_name="c", subcore_axis_name="s")

    @functools.partial(
        pl.kernel,
        mesh=mesh,
        compiler_params=pltpu.CompilerParams(
            needs_layout_passes=False, use_tc_tiling_on_sc=False),
        out_type=jax.ShapeDtypeStruct((B,), jnp.float32),
        scratch_types=[
            pltpu.VMEM((n_per_w,), jnp.int32),
            pltpu.VMEM((n_per_w,), jnp.int32),
            pltpu.VMEM((n_per_w,), jnp.int32),
            pltpu.VMEM((n_per_w, _L), jnp.float32),
            pltpu.VMEM((n_per_w, _L), jnp.float32),
            pltpu.VMEM((n_per_w,), jnp.float32),
            pltpu.SemaphoreType.DMA,
            pltpu.SemaphoreType.DMA,
        ],
    )
    def k(layer_hbm, aidx_hbm, bidx_hbm, a_hbm, b_hbm, out_hbm,
          layer_v, aidx_v, bidx_v, arows_v, brows_v, out_v, sem_a, sem_b):
        wid = lax.axis_index("s") * NC + lax.axis_index("c")
        base = wid * n_per_w

        pltpu.sync_copy(layer_hbm.at[pl.ds(base, n_per_w)], layer_v)
        pltpu.sync_copy(aidx_hbm.at[pl.ds(base, n_per_w)], aidx_v)
        pltpu.sync_copy(bidx_hbm.at[pl.ds(base, n_per_w)], bidx_v)

        def idx_body(g, _):
            off = g * _L
            sl = pl.ds(off, _L)
            aidx_v[sl] = layer_v[sl] * NUM + aidx_v[sl]
            return 0

        lax.fori_loop(0, n_per_w // _L, idx_body, 0)

        cp_a = pltpu.async_copy(a_hbm.at[aidx_v], arows_v, sem_a)
        cp_b = pltpu.async_copy(b_hbm.at[bidx_v], brows_v, sem_b)
        cp_a.wait()
        cp_b.wait()

        lanes = lax.iota(jnp.int32, _L)

        def dot_body(g, _):
            off = g * _L
            rows = off + lanes
            acc = jnp.zeros((_L,), jnp.float32)
            for d in range(_L):
                cols = (lanes + d) & (_L - 1)
                av = plsc.load_gather(arows_v, [rows, cols])
                bv = plsc.load_gather(brows_v, [rows, cols])
                acc = acc + av * bv
            out_v[pl.ds(off, _L)] = acc
            return 0

        lax.fori_loop(0, n_per_w // _L, dot_body, 0)

        pltpu.sync_copy(out_v, out_hbm.at[pl.ds(base, n_per_w)])

    return k


def kernel(layerIdx, aIdx, bIdx, A_table, B_table):
    B = layerIdx.shape[0]
    NUM, D = B_table.shape
    k = _build(B, D, NUM, A_table.shape[0])
    return k(layerIdx.astype(jnp.int32), aIdx.astype(jnp.int32),
             bIdx.astype(jnp.int32), A_table, B_table)


# speedup vs baseline: 2.5675x; 2.5675x over previous
import functools

import jax
import jax.numpy as jnp
from jax import lax
from jax.experimental import pallas as pl
from jax.experimental.pallas import tpu as pltpu
from jax.experimental.pallas import tpu_sc as plsc

_L = 16


@functools.lru_cache(maxsize=None)
def _build(B, D, NUM, n_rows_a):
    info = plsc.get_sparse_core_info()
    NC, NS = info.num_cores, info.num_subcores
    NW = NC * NS
    assert B % (8 * NW) == 0 and D == _L
    n_per_w = B // NW
    n_groups = n_per_w // _L
    n_flat = n_per_w * D

    mesh = plsc.VectorSubcoreMesh(core_axis_name="c", subcore_axis_name="s")

    @functools.partial(
        pl.kernel,
        mesh=mesh,
        compiler_params=pltpu.CompilerParams(
            needs_layout_passes=False, use_tc_tiling_on_sc=False),
        out_type=jax.ShapeDtypeStruct((B,), jnp.float32),
        scratch_types=[
            pltpu.VMEM((n_per_w,), jnp.int32),
            pltpu.VMEM((n_per_w,), jnp.int32),
            pltpu.VMEM((n_per_w,), jnp.int32),
            pltpu.VMEM((n_flat,), jnp.int32),
            pltpu.VMEM((n_flat,), jnp.int32),
            pltpu.VMEM((n_flat,), jnp.float32),
            pltpu.VMEM((n_flat,), jnp.float32),
            pltpu.VMEM((n_per_w,), jnp.float32),
            pltpu.SemaphoreType.DMA,
            pltpu.SemaphoreType.DMA,
        ],
    )
    def k(layer_hbm, aidx_hbm, bidx_hbm, a_hbm, b_hbm, out_hbm,
          layer_v, aidx_v, bidx_v, idxa_v, idxb_v, arows_v, brows_v, out_v,
          sem_a, sem_b):
        wid = lax.axis_index("s") * NC + lax.axis_index("c")
        base = wid * n_per_w

        pltpu.sync_copy(layer_hbm.at[pl.ds(base, n_per_w)], layer_v)
        pltpu.sync_copy(aidx_hbm.at[pl.ds(base, n_per_w)], aidx_v)
        pltpu.sync_copy(bidx_hbm.at[pl.ds(base, n_per_w)], bidx_v)

        def idx_body(g, _):
            off = g * _L
            sl = pl.ds(off, _L)
            r = layer_v[sl] * NUM + aidx_v[sl]
            jc = ((r >> 7) << 10) + (r & 127)
            b16 = bidx_v[sl] << 4
            for d in range(_L):
                ca = ((d >> 3) * (8 * n_rows_a)) + ((d & 7) << 7)
                idxa_v[pl.ds(d * n_per_w + off, _L)] = jc + ca
                idxb_v[pl.ds(d * n_per_w + off, _L)] = b16 + d
            return 0

        lax.fori_loop(0, n_groups, idx_body, 0)

        cp_a = pltpu.async_copy(a_hbm.at[idxa_v], arows_v, sem_a)
        cp_b = pltpu.async_copy(b_hbm.at[idxb_v], brows_v, sem_b)
        cp_a.wait()
        cp_b.wait()

        def dot_body(g, _):
            off = g * _L
            acc = jnp.zeros((_L,), jnp.float32)
            for d in range(_L):
                sl = pl.ds(d * n_per_w + off, _L)
                acc = acc + arows_v[sl] * brows_v[sl]
            out_v[pl.ds(off, _L)] = acc
            return 0

        lax.fori_loop(0, n_groups, dot_body, 0)

        pltpu.sync_copy(out_v, out_hbm.at[pl.ds(base, n_per_w)])

    return k


def kernel(layerIdx, aIdx, bIdx, A_table, B_table):
    B = layerIdx.shape[0]
    NUM, D = B_table.shape
    n_rows_a = A_table.shape[0]
    assert n_rows_a % 128 == 0 and D % 8 == 0
    a_flat = A_table.reshape(n_rows_a // 128, 128, D // 8, 8)
    a_flat = a_flat.transpose(2, 0, 3, 1).reshape(-1)
    b_flat = B_table.reshape(-1)
    k = _build(B, D, NUM, n_rows_a)
    return k(layerIdx.astype(jnp.int32), aIdx.astype(jnp.int32),
             bIdx.astype(jnp.int32), a_flat, b_flat)


# speedup vs baseline: 2.9054x; 1.1316x over previous
import functools

import jax
import jax.numpy as jnp
from jax import lax
from jax.experimental import pallas as pl
from jax.experimental.pallas import tpu as pltpu
from jax.experimental.pallas import tpu_sc as plsc

_L = 16


@functools.lru_cache(maxsize=None)
def _build(B, D, NUM, n_rows_a):
    info = plsc.get_sparse_core_info()
    NC, NS = info.num_cores, info.num_subcores
    NW = NC * NS
    assert B % (8 * NW) == 0 and D == _L
    n_per_w = B // NW
    n_groups = n_per_w // _L
    n_flat = n_per_w * D

    mesh = plsc.VectorSubcoreMesh(core_axis_name="c", subcore_axis_name="s")

    @functools.partial(
        pl.kernel,
        mesh=mesh,
        compiler_params=pltpu.CompilerParams(
            needs_layout_passes=False, use_tc_tiling_on_sc=False),
        out_type=jax.ShapeDtypeStruct((B,), jnp.float32),
        scratch_types=[
            pltpu.VMEM((n_per_w,), jnp.int32),
            pltpu.VMEM((n_per_w,), jnp.int32),
            pltpu.VMEM((n_per_w,), jnp.int32),
            pltpu.VMEM((n_flat,), jnp.int32),
            pltpu.VMEM((n_flat,), jnp.float32),
            pltpu.VMEM((n_per_w, _L), jnp.float32),
            pltpu.VMEM((n_per_w,), jnp.float32),
            pltpu.SemaphoreType.DMA,
            pltpu.SemaphoreType.DMA,
        ],
    )
    def k(layer_hbm, aidx_hbm, bidx_hbm, a_hbm, b_hbm, out_hbm,
          layer_v, aidx_v, bidx_v, idxa_v, arows_v, brows_v, out_v,
          sem_a, sem_b):
        wid = lax.axis_index("s") * NC + lax.axis_index("c")
        base = wid * n_per_w

        pltpu.sync_copy(bidx_hbm.at[pl.ds(base, n_per_w)], bidx_v)
        cp_b = pltpu.async_copy(b_hbm.at[bidx_v], brows_v, sem_b)

        pltpu.sync_copy(layer_hbm.at[pl.ds(base, n_per_w)], layer_v)
        pltpu.sync_copy(aidx_hbm.at[pl.ds(base, n_per_w)], aidx_v)

        lanes = lax.iota(jnp.int32, _L)
        diag = [(lanes + d) & (_L - 1) for d in range(_L)]
        ca = [((dg >> 3) * (8 * n_rows_a)) + ((dg & 7) << 7) for dg in diag]

        def idx_body(g, _):
            off = g * _L
            sl = pl.ds(off, _L)
            r = layer_v[sl] * NUM + aidx_v[sl]
            jc = ((r >> 7) << 10) + (r & 127)
            for d in range(_L):
                idxa_v[pl.ds(d * n_per_w + off, _L)] = jc + ca[d]
            return 0

        lax.fori_loop(0, n_groups, idx_body, 0)

        cp_a = pltpu.async_copy(a_hbm.at[idxa_v], arows_v, sem_a)
        cp_a.wait()
        cp_b.wait()

        def dot_body(g, _):
            off = g * _L
            rows = off + lanes
            acc = jnp.zeros((_L,), jnp.float32)
            for d in range(_L):
                av = arows_v[pl.ds(d * n_per_w + off, _L)]
                bv = plsc.load_gather(brows_v, [rows, diag[d]])
                acc = acc + av * bv
            out_v[pl.ds(off, _L)] = acc
            return 0

        lax.fori_loop(0, n_groups, dot_body, 0)

        pltpu.sync_copy(out_v, out_hbm.at[pl.ds(base, n_per_w)])

    return k


def kernel(layerIdx, aIdx, bIdx, A_table, B_table):
    B = layerIdx.shape[0]
    NUM, D = B_table.shape
    n_rows_a = A_table.shape[0]
    assert n_rows_a % 128 == 0 and D % 8 == 0
    a_flat = A_table.reshape(n_rows_a // 128, 128, D // 8, 8)
    a_flat = a_flat.transpose(2, 0, 3, 1).reshape(-1)
    k = _build(B, D, NUM, n_rows_a)
    return k(layerIdx.astype(jnp.int32), aIdx.astype(jnp.int32),
             bIdx.astype(jnp.int32), a_flat, B_table)


# speedup vs baseline: 5.0009x; 1.7213x over previous
import functools

import jax
import jax.numpy as jnp
from jax import lax
from jax.experimental import pallas as pl
from jax.experimental.pallas import tpu as pltpu
from jax.experimental.pallas import tpu_sc as plsc

_L = 16


def _flat_view(table):
    n, d = table.shape
    t = table.reshape(n // 128, 128, d // 8, 8)
    return t.transpose(2, 0, 3, 1).reshape(-1)


@functools.lru_cache(maxsize=None)
def _build(B, D, NUM, n_rows_a, n_rows_b):
    info = plsc.get_sparse_core_info()
    NC, NS = info.num_cores, info.num_subcores
    NW = NC * NS
    assert B % (8 * NW) == 0 and D == _L
    n_per_w = B // NW
    n_groups = n_per_w // _L
    n_flat = n_per_w * D

    mesh = plsc.VectorSubcoreMesh(core_axis_name="c", subcore_axis_name="s")

    @functools.partial(
        pl.kernel,
        mesh=mesh,
        compiler_params=pltpu.CompilerParams(
            needs_layout_passes=False, use_tc_tiling_on_sc=False),
        out_type=jax.ShapeDtypeStruct((B,), jnp.float32),
        scratch_types=[
            pltpu.VMEM((n_per_w,), jnp.int32),
            pltpu.VMEM((n_per_w,), jnp.int32),
            pltpu.VMEM((n_per_w,), jnp.int32),
            pltpu.VMEM((n_flat,), jnp.int32),
            pltpu.VMEM((n_flat,), jnp.int32),
            pltpu.VMEM((n_flat,), jnp.float32),
            pltpu.VMEM((n_flat,), jnp.float32),
            pltpu.VMEM((n_per_w,), jnp.float32),
            pltpu.SemaphoreType.DMA,
            pltpu.SemaphoreType.DMA,
        ],
    )
    def k(layer_hbm, aidx_hbm, bidx_hbm, a_hbm, b_hbm, out_hbm,
          layer_v, aidx_v, bidx_v, idxa_v, idxb_v, arows_v, brows_v, out_v,
          sem_a, sem_b):
        wid = lax.axis_index("s") * NC + lax.axis_index("c")
        base = wid * n_per_w

        pltpu.sync_copy(layer_hbm.at[pl.ds(base, n_per_w)], layer_v)
        pltpu.sync_copy(aidx_hbm.at[pl.ds(base, n_per_w)], aidx_v)
        pltpu.sync_copy(bidx_hbm.at[pl.ds(base, n_per_w)], bidx_v)

        def idx_body(g, _):
            off = g * _L
            sl = pl.ds(off, _L)
            ra = layer_v[sl] * NUM + aidx_v[sl]
            rb = bidx_v[sl]
            jca = ((ra >> 7) << 10) + (ra & 127)
            jcb = ((rb >> 7) << 10) + (rb & 127)
            for d in range(_L):
                ca = ((d >> 3) * (8 * n_rows_a)) + ((d & 7) << 7)
                cb = ((d >> 3) * (8 * n_rows_b)) + ((d & 7) << 7)
                idxa_v[pl.ds(d * n_per_w + off, _L)] = jca + ca
                idxb_v[pl.ds(d * n_per_w + off, _L)] = jcb + cb
            return 0

        lax.fori_loop(0, n_groups, idx_body, 0)

        cp_a = pltpu.async_copy(a_hbm.at[idxa_v], arows_v, sem_a)
        cp_b = pltpu.async_copy(b_hbm.at[idxb_v], brows_v, sem_b)
        cp_a.wait()
        cp_b.wait()

        def dot_body(g, _):
            off = g * _L
            acc = jnp.zeros((_L,), jnp.float32)
            for d in range(_L):
                sl = pl.ds(d * n_per_w + off, _L)
                acc = acc + arows_v[sl] * brows_v[sl]
            out_v[pl.ds(off, _L)] = acc
            return 0

        lax.fori_loop(0, n_groups, dot_body, 0)

        pltpu.sync_copy(out_v, out_hbm.at[pl.ds(base, n_per_w)])

    return k


def kernel(layerIdx, aIdx, bIdx, A_table, B_table):
    B = layerIdx.shape[0]
    NUM, D = B_table.shape
    n_rows_a = A_table.shape[0]
    assert n_rows_a % 128 == 0 and D % 8 == 0
    pad_b = (-NUM) % 128
    b_padded = jnp.pad(B_table, ((0, pad_b), (0, 0)))
    a_flat = _flat_view(A_table)
    b_flat = _flat_view(b_padded)
    k = _build(B, D, NUM, n_rows_a, NUM + pad_b)
    return k(layerIdx.astype(jnp.int32), aIdx.astype(jnp.int32),
             bIdx.astype(jnp.int32), a_flat, b_flat)
